# Initial kernel scaffold; baseline (speedup 1.0000x reference)
import jax
import jax.numpy as jnp
from jax.experimental import pallas as pl


def kernel(x, edge_index, edge_attr, params):
    raise NotImplementedError("write your pallas kernel here")


# speedup vs baseline: 3.9649x; 3.9649x over previous
import functools
import jax
import jax.numpy as jnp
from jax import lax
from jax.experimental import pallas as pl
from jax.experimental.pallas import tpu as pltpu, tpu_sc as plsc

NN = 10000
NP = 10240
EE = 160000
DD = 512
HH = 4
BM = 512
BE = 2000
BA = 80
BB = 40
RPT = NP // 16

_MESH = dict(core_axis_name="c", subcore_axis_name="s")


def _ln(z, g, b):
    mu = jnp.mean(z, axis=-1, keepdims=True)
    var = jnp.mean((z - mu) ** 2, axis=-1, keepdims=True)
    return (z - mu) * jax.lax.rsqrt(var + 1e-5) * g + b


def _cat4(ref):
    return jnp.concatenate([ref[0], ref[1], ref[2], ref[3]], axis=-1)


def _tc_inproj(xp, wT, b):
    def body(x_ref, w_ref, b_ref, o_ref):
        z = jnp.maximum(
            jnp.dot(x_ref[...], w_ref[...],
                    preferred_element_type=jnp.float32) + b_ref[0], 0.0)
        o_ref[...] = z[None]

    return pl.pallas_call(
        body,
        grid=(NP // BM, 4),
        in_specs=[
            pl.BlockSpec((BM, 128), lambda i, j: (i, 0)),
            pl.BlockSpec((128, 128), lambda i, j: (0, j)),
            pl.BlockSpec((1, 1, 128), lambda i, j: (j, 0, 0)),
        ],
        out_specs=pl.BlockSpec((1, BM, 128), lambda i, j: (j, i, 0)),
        out_shape=jax.ShapeDtypeStruct((4, NP, 128), jnp.float32),
    )(xp, wT, b)


def _tc_edge_mm(ea8, wT, b, ng):
    def body(a_ref, w_ref, b_ref, o_ref):
        z = jnp.dot(a_ref[...], w_ref[...],
                    preferred_element_type=jnp.float32) + b_ref[0]
        o_ref[...] = z[None]

    return pl.pallas_call(
        body,
        grid=(EE // BE, ng),
        in_specs=[
            pl.BlockSpec((BE, 8), lambda i, j: (i, 0)),
            pl.BlockSpec((8, 128), lambda i, j: (0, j)),
            pl.BlockSpec((1, 1, 128), lambda i, j: (j, 0, 0)),
        ],
        out_specs=pl.BlockSpec((1, BE, 128), lambda i, j: (j, i, 0)),
        out_shape=jax.ShapeDtypeStruct((ng, EE, 128), jnp.float32),
    )(ea8, wT, b)


def _tc_gine_mlp(h0c, agg4, w1T, b1, w2T, b2, lng, lnb):
    def body(h_ref, a_ref, w1_ref, b1_ref, w2_ref, b2_ref, g_ref, be_ref,
             o_ref):
        h0 = _cat4(h_ref)
        z = h0 + _cat4(a_ref)
        z = jnp.maximum(
            jnp.dot(z, w1_ref[...], preferred_element_type=jnp.float32)
            + b1_ref[...], 0.0)
        z = jnp.dot(z, w2_ref[...],
                    preferred_element_type=jnp.float32) + b2_ref[...]
        z = _ln(jnp.maximum(z, 0.0) + h0, g_ref[...], be_ref[...])
        for c in range(4):
            o_ref[c] = z[:, c * 128:(c + 1) * 128]

    return pl.pallas_call(
        body,
        grid=(NP // BM,),
        in_specs=[
            pl.BlockSpec((4, BM, 128), lambda i: (0, i, 0)),
            pl.BlockSpec((4, BM, 128), lambda i: (0, i, 0)),
            pl.BlockSpec((DD, 2 * DD), lambda i: (0, 0)),
            pl.BlockSpec((1, 2 * DD), lambda i: (0, 0)),
            pl.BlockSpec((2 * DD, DD), lambda i: (0, 0)),
            pl.BlockSpec((1, DD), lambda i: (0, 0)),
            pl.BlockSpec((1, DD), lambda i: (0, 0)),
            pl.BlockSpec((1, DD), lambda i: (0, 0)),
        ],
        out_specs=pl.BlockSpec((4, BM, 128), lambda i: (0, i, 0)),
        out_shape=jax.ShapeDtypeStruct((4, NP, 128), jnp.float32),
    )(h0c, agg4, w1T, b1, w2T, b2, lng, lnb)


def _tc_gat_lin(h1c, wlT, wrT, bl, br):
    def body(h_ref, wl_ref, wr_ref, bl_ref, br_ref, ol_ref, or_ref):
        h1 = _cat4(h_ref)
        ol_ref[...] = (jnp.dot(h1, wl_ref[...],
                               preferred_element_type=jnp.float32)
                       + bl_ref[0])[None]
        or_ref[...] = (jnp.dot(h1, wr_ref[...],
                               preferred_element_type=jnp.float32)
                       + br_ref[0])[None]

    return pl.pallas_call(
        body,
        grid=(NP // BM, 16),
        in_specs=[
            pl.BlockSpec((4, BM, 128), lambda i, j: (0, i, 0)),
            pl.BlockSpec((DD, 128), lambda i, j: (0, j)),
            pl.BlockSpec((DD, 128), lambda i, j: (0, j)),
            pl.BlockSpec((1, 1, 128), lambda i, j: (j, 0, 0)),
            pl.BlockSpec((1, 1, 128), lambda i, j: (j, 0, 0)),
        ],
        out_specs=[
            pl.BlockSpec((1, BM, 128), lambda i, j: (j, i, 0)),
            pl.BlockSpec((1, BM, 128), lambda i, j: (j, i, 0)),
        ],
        out_shape=[
            jax.ShapeDtypeStruct((16, NP, 128), jnp.float32),
            jax.ShapeDtypeStruct((16, NP, 128), jnp.float32),
        ],
    )(h1c, wlT, wrT, bl, br)


def _tc_logit_max(logit16):
    def body(l_ref, o_ref):
        i = pl.program_id(0)
        lane = lax.broadcasted_iota(jnp.int32, (BE, 16), 1)
        m = jnp.max(jnp.where(lane < HH, l_ref[...], -3e38))

        @pl.when(i == 0)
        def _():
            o_ref[0, 0] = m

        @pl.when(i > 0)
        def _():
            o_ref[0, 0] = jnp.maximum(o_ref[0, 0], m)

    return pl.pallas_call(
        body,
        grid=(EE // BE,),
        in_specs=[pl.BlockSpec((BE, 16), lambda i: (i, 0))],
        out_specs=pl.BlockSpec(memory_space=pltpu.SMEM),
        out_shape=jax.ShapeDtypeStruct((1, 1), jnp.float32),
    )(logit16)


def _tc_exp(logit16, cmax):
    def body(l_ref, c_ref, oe_ref, oh_ref):
        c = c_ref[0, 0]
        lane = lax.broadcasted_iota(jnp.int32, (BE, 16), 1)
        ex = jnp.exp(l_ref[...] - c)
        oe_ref[...] = jnp.where(lane < HH, ex,
                                jnp.where(lane == HH, 1.0, 0.0))
        for h in range(HH):
            oh_ref[h] = jnp.broadcast_to(ex[:, h:h + 1], (BE, 16))

    return pl.pallas_call(
        body,
        grid=(EE // BE,),
        in_specs=[
            pl.BlockSpec((BE, 16), lambda i: (i, 0)),
            pl.BlockSpec(memory_space=pltpu.SMEM),
        ],
        out_specs=[
            pl.BlockSpec((BE, 16), lambda i: (i, 0)),
            pl.BlockSpec((4, BE, 16), lambda i: (0, i, 0)),
        ],
        out_shape=[
            jax.ShapeDtypeStruct((EE, 16), jnp.float32),
            jax.ShapeDtypeStruct((HH, EE, 16), jnp.float32),
        ],
    )(logit16, cmax)


def _tc_denom_finish(denom2):
    def body(d_ref, rcp_ref, dv_ref):
        d = d_ref[0] + d_ref[1]
        for h in range(HH):
            rcp_ref[h] = jnp.broadcast_to(
                1.0 / (d[:, h:h + 1] + 1e-16), (BM, 16))
        deg = d[:, HH:HH + 1] + 1.0
        dv_ref[...] = jnp.broadcast_to(jax.lax.rsqrt(deg), (BM, 16))

    return pl.pallas_call(
        body,
        grid=(NP // BM,),
        in_specs=[pl.BlockSpec((2, BM, 16), lambda i: (0, i, 0))],
        out_specs=[
            pl.BlockSpec((4, BM, 16), lambda i: (0, i, 0)),
            pl.BlockSpec((BM, 16), lambda i: (i, 0)),
        ],
        out_shape=[
            jax.ShapeDtypeStruct((HH, NP, 16), jnp.float32),
            jax.ShapeDtypeStruct((NP, 16), jnp.float32),
        ],
    )(denom2)


def _tc_gat_finish(gat16, rcph, dinv16, h1c, bias, lng, lnb, wT):
    def body(g16_ref, rcp_ref, dv_ref, h_ref, b_ref, g_ref, be_ref, w_ref,
             oh_ref, ox_ref):
        cs = []
        for c in range(4):
            s = (g16_ref[c] * rcp_ref[0][:, 0:1]
                 + g16_ref[4 + c] * rcp_ref[1][:, 0:1]
                 + g16_ref[8 + c] * rcp_ref[2][:, 0:1]
                 + g16_ref[12 + c] * rcp_ref[3][:, 0:1])
            cs.append(0.25 * s)
        z = jnp.concatenate(cs, axis=-1) + b_ref[...]
        res = _cat4(h_ref)
        h2 = _ln(jnp.maximum(z, 0.0) + res, g_ref[...], be_ref[...])
        xw = dv_ref[:, 0:1] * jnp.dot(h2, w_ref[...],
                                      preferred_element_type=jnp.float32)
        for c in range(4):
            oh_ref[c] = h2[:, c * 128:(c + 1) * 128]
            ox_ref[c] = xw[:, c * 128:(c + 1) * 128]

    return pl.pallas_call(
        body,
        grid=(NP // BM,),
        in_specs=[
            pl.BlockSpec((16, BM, 128), lambda i: (0, i, 0)),
            pl.BlockSpec((4, BM, 16), lambda i: (0, i, 0)),
            pl.BlockSpec((BM, 16), lambda i: (i, 0)),
            pl.BlockSpec((4, BM, 128), lambda i: (0, i, 0)),
            pl.BlockSpec((1, DD), lambda i: (0, 0)),
            pl.BlockSpec((1, DD), lambda i: (0, 0)),
            pl.BlockSpec((1, DD), lambda i: (0, 0)),
            pl.BlockSpec((DD, DD), lambda i: (0, 0)),
        ],
        out_specs=[
            pl.BlockSpec((4, BM, 128), lambda i: (0, i, 0)),
            pl.BlockSpec((4, BM, 128), lambda i: (0, i, 0)),
        ],
        out_shape=[
            jax.ShapeDtypeStruct((4, NP, 128), jnp.float32),
            jax.ShapeDtypeStruct((4, NP, 128), jnp.float32),
        ],
    )(gat16, rcph, dinv16, h1c, bias, lng, lnb, wT)


def _tc_gcn_finish(agg4, xwc, dinv16, hres, bias, lng, lnb, wT, make_xw):
    def body(a_ref, x_ref, d_ref, h_ref, b_ref, g_ref, be_ref, w_ref,
             oh_ref, ox_ref):
        d1 = d_ref[:, 0:1]
        z = d1 * (_cat4(a_ref) + _cat4(x_ref)) + b_ref[...]
        res = _cat4(h_ref)
        hn = _ln(jnp.maximum(z, 0.0) + res, g_ref[...], be_ref[...])
        for c in range(4):
            oh_ref[c] = hn[:, c * 128:(c + 1) * 128]
        if make_xw:
            xw = d1 * jnp.dot(hn, w_ref[...],
                              preferred_element_type=jnp.float32)
            for c in range(4):
                ox_ref[c] = xw[:, c * 128:(c + 1) * 128]
        else:
            ox_ref[0] = jnp.zeros((BM, 128), jnp.float32)

    return pl.pallas_call(
        body,
        grid=(NP // BM,),
        in_specs=[
            pl.BlockSpec((4, BM, 128), lambda i: (0, i, 0)),
            pl.BlockSpec((4, BM, 128), lambda i: (0, i, 0)),
            pl.BlockSpec((BM, 16), lambda i: (i, 0)),
            pl.BlockSpec((4, BM, 128), lambda i: (0, i, 0)),
            pl.BlockSpec((1, DD), lambda i: (0, 0)),
            pl.BlockSpec((1, DD), lambda i: (0, 0)),
            pl.BlockSpec((1, DD), lambda i: (0, 0)),
            pl.BlockSpec((DD, DD), lambda i: (0, 0)),
        ],
        out_specs=[
            pl.BlockSpec((4, BM, 128), lambda i: (0, i, 0)),
            pl.BlockSpec((4, BM, 128) if make_xw else (1, BM, 128),
                         (lambda i: (0, i, 0)) if make_xw
                         else (lambda i: (0, i, 0))),
        ],
        out_shape=[
            jax.ShapeDtypeStruct((4, NP, 128), jnp.float32),
            jax.ShapeDtypeStruct((4, NP, 128) if make_xw else (1, NP, 128),
                                 jnp.float32),
        ],
    )(agg4, xwc, dinv16, hres, bias, lng, lnb, wT)


def _sc_gine_agg(h0c, elinc, srci, dsti, zrows):
    mesh = plsc.VectorSubcoreMesh(**_MESH)

    @functools.partial(
        pl.kernel,
        out_type=jax.ShapeDtypeStruct((4, NP, 128), jnp.float32),
        mesh=mesh,
        scratch_types=[
            pltpu.VMEM((BA,), jnp.int32),
            pltpu.VMEM((BA,), jnp.int32),
            pltpu.VMEM((BA, 128), jnp.float32),
            pltpu.VMEM((BA, 128), jnp.float32),
            pltpu.VMEM_SHARED((NP, 128), jnp.float32),
            pltpu.SemaphoreType.DMA,
        ],
    )
    def k(h_h, e_h, s_h, d_h, z_h, o_h, idxv, didxv, gbuf, ebuf, acc, sem):
        cid = lax.axis_index("c")
        sid = lax.axis_index("s")
        for kk in range(2):
            chunk = cid * 2 + kk
            pltpu.sync_copy(z_h.at[pl.ds(sid * RPT, RPT)],
                            acc.at[pl.ds(sid * RPT, RPT)])
            plsc.subcore_barrier()

            def blk(b, _):
                e0 = sid * 10000 + b * BA
                c1 = pltpu.async_copy(s_h.at[pl.ds(e0, BA)], idxv, sem)
                c2 = pltpu.async_copy(d_h.at[pl.ds(e0, BA)], didxv, sem)
                c3 = pltpu.async_copy(
                    e_h.at[chunk].at[pl.ds(e0, BA)], ebuf, sem)
                c1.wait()
                c4 = pltpu.async_copy(h_h.at[chunk].at[idxv], gbuf, sem)
                c2.wait()
                c3.wait()
                c4.wait()

                def rbody(r, _):
                    for v in range(8):
                        sl = pl.ds(v * 16, 16)
                        gbuf[r, sl] = jnp.maximum(gbuf[r, sl] + ebuf[r, sl],
                                                  0.0)
                    return 0

                lax.fori_loop(0, BA, rbody, 0)
                pltpu.sync_copy(gbuf, acc.at[didxv], add=True)
                return 0

            lax.fori_loop(0, EE // 16 // BA, blk, 0)
            plsc.subcore_barrier()
            pltpu.sync_copy(acc.at[pl.ds(sid * RPT, RPT)],
                            o_h.at[chunk].at[pl.ds(sid * RPT, RPT)])
            plsc.subcore_barrier()

    return k(h0c, elinc, srci, dsti, zrows)


def _sc_gat_logits(xlg, xrg, eeg, att16, srci, dsti):
    mesh = plsc.VectorSubcoreMesh(**_MESH)
    nb = EE // 32 // BB

    @functools.partial(
        pl.kernel,
        out_type=jax.ShapeDtypeStruct((EE, 16), jnp.float32),
        mesh=mesh,
        scratch_types=[
            pltpu.VMEM((BB,), jnp.int32),
            pltpu.VMEM((BB,), jnp.int32),
            [pltpu.VMEM((BB, 128), jnp.float32) for _ in range(4)],
            [pltpu.VMEM((BB, 128), jnp.float32) for _ in range(4)],
            [pltpu.VMEM((BB, 128), jnp.float32) for _ in range(4)],
            pltpu.VMEM((16, 128), jnp.float32),
            pltpu.VMEM((BB, 16), jnp.float32),
            pltpu.SemaphoreType.DMA,
        ],
    )
    def k(xl_h, xr_h, ee_h, att_h, s_h, d_h, o_h, idxv, didxv, xlb, xrb,
          eeb, attb, lbuf, sem):
        cid = lax.axis_index("c")
        sid = lax.axis_index("s")
        wid = cid * 16 + sid
        pltpu.sync_copy(att_h, attb)
        lanes = lax.iota(jnp.int32, 16)

        def blk(b, _):
            e0 = wid * (EE // 32) + b * BB
            c1 = pltpu.async_copy(s_h.at[pl.ds(e0, BB)], idxv, sem)
            c2 = pltpu.async_copy(d_h.at[pl.ds(e0, BB)], didxv, sem)
            c1.wait()
            c2.wait()
            for h in range(HH):
                cps = []
                for c in range(4):
                    g = h * 4 + c
                    cps.append(pltpu.async_copy(
                        xl_h.at[g].at[idxv], xlb[c], sem))
                    cps.append(pltpu.async_copy(
                        xr_h.at[g].at[didxv], xrb[c], sem))
                    cps.append(pltpu.async_copy(
                        ee_h.at[g].at[pl.ds(e0, BB)], eeb[c], sem))
                for cp in cps:
                    cp.wait()

                def rbody(r, _):
                    accv = jnp.zeros((16,), jnp.float32)
                    for c in range(4):
                        for v in range(8):
                            sl = pl.ds(v * 16, 16)
                            s = xlb[c][r, sl] + xrb[c][r, sl] + eeb[c][r, sl]
                            m = jnp.maximum(s, 0.0) + 0.2 * jnp.minimum(
                                s, 0.0)
                            accv = accv + m * attb[h * 4 + c, sl]
                    for sh in (8, 4, 2, 1):
                        accv = accv + accv.at[lanes ^ sh].get(
                            mode="promise_in_bounds")
                    if h == 0:
                        row = jnp.where(lanes == 0, accv, 0.0)
                    else:
                        row = jnp.where(lanes == h, accv,
                                        lbuf[r, pl.ds(0, 16)])
                    lbuf[r, pl.ds(0, 16)] = row
                    return 0

                lax.fori_loop(0, BB, rbody, 0)
            pltpu.sync_copy(lbuf, o_h.at[pl.ds(e0, BB)])
            return 0

        lax.fori_loop(0, nb, blk, 0)

    return k(xlg, xrg, eeg, att16, srci, dsti)


def _sc_denom(ex16, dsti, zrows16):
    mesh = plsc.VectorSubcoreMesh(**_MESH)
    nb = EE // 32 // BB

    @functools.partial(
        pl.kernel,
        out_type=jax.ShapeDtypeStruct((2, NP, 16), jnp.float32),
        mesh=mesh,
        scratch_types=[
            pltpu.VMEM((BB,), jnp.int32),
            pltpu.VMEM((BB, 16), jnp.float32),
            pltpu.VMEM_SHARED((NP, 16), jnp.float32),
            pltpu.SemaphoreType.DMA,
        ],
    )
    def k(ex_h, d_h, z_h, o_h, didxv, exb, acc, sem):
        cid = lax.axis_index("c")
        sid = lax.axis_index("s")
        pltpu.sync_copy(z_h.at[pl.ds(sid * RPT, RPT)],
                        acc.at[pl.ds(sid * RPT, RPT)])
        plsc.subcore_barrier()

        def blk(b, _):
            e0 = (cid * 16 + sid) * (EE // 32) + b * BB
            c1 = pltpu.async_copy(d_h.at[pl.ds(e0, BB)], didxv, sem)
            c2 = pltpu.async_copy(ex_h.at[pl.ds(e0, BB)], exb, sem)
            c1.wait()
            c2.wait()
            pltpu.sync_copy(exb, acc.at[didxv], add=True)
            return 0

        lax.fori_loop(0, nb, blk, 0)
        plsc.subcore_barrier()
        pltpu.sync_copy(acc.at[pl.ds(sid * RPT, RPT)],
                        o_h.at[cid].at[pl.ds(sid * RPT, RPT)])

    return k(ex16, dsti, zrows16)


def _sc_gat_out(xlg, exh, srci, dsti, zrows):
    mesh = plsc.VectorSubcoreMesh(**_MESH)

    @functools.partial(
        pl.kernel,
        out_type=jax.ShapeDtypeStruct((16, NP, 128), jnp.float32),
        mesh=mesh,
        scratch_types=[
            pltpu.VMEM((BA,), jnp.int32),
            pltpu.VMEM((BA,), jnp.int32),
            pltpu.VMEM((BA, 128), jnp.float32),
            pltpu.VMEM((BA, 16), jnp.float32),
            pltpu.VMEM_SHARED((NP, 128), jnp.float32),
            pltpu.SemaphoreType.DMA,
        ],
    )
    def k(xl_h, ex_h, s_h, d_h, z_h, o_h, idxv, didxv, gbuf, exb, acc, sem):
        cid = lax.axis_index("c")
        sid = lax.axis_index("s")
        for g8 in range(8):
            g = cid * 8 + g8
            h = cid * 2 + (g8 // 4)
            pltpu.sync_copy(z_h.at[pl.ds(sid * RPT, RPT)],
                            acc.at[pl.ds(sid * RPT, RPT)])
            plsc.subcore_barrier()

            def blk(b, _):
                e0 = sid * 10000 + b * BA
                c1 = pltpu.async_copy(s_h.at[pl.ds(e0, BA)], idxv, sem)
                c2 = pltpu.async_copy(d_h.at[pl.ds(e0, BA)], didxv, sem)
                c3 = pltpu.async_copy(ex_h.at[h].at[pl.ds(e0, BA)], exb, sem)
                c1.wait()
                c2.wait()
                c4 = pltpu.async_copy(xl_h.at[g].at[idxv], gbuf, sem)
                c3.wait()
                c4.wait()

                def rbody(r, _):
                    av = exb[r, pl.ds(0, 16)]
                    for v in range(8):
                        sl = pl.ds(v * 16, 16)
                        gbuf[r, sl] = gbuf[r, sl] * av
                    return 0

                lax.fori_loop(0, BA, rbody, 0)
                pltpu.sync_copy(gbuf, acc.at[didxv], add=True)
                return 0

            lax.fori_loop(0, EE // 16 // BA, blk, 0)
            plsc.subcore_barrier()
            pltpu.sync_copy(acc.at[pl.ds(sid * RPT, RPT)],
                            o_h.at[g].at[pl.ds(sid * RPT, RPT)])
            plsc.subcore_barrier()

    return k(xlg, exh, srci, dsti, zrows)


def _sc_gcn_agg(xwc, srci, dsti, zrows):
    mesh = plsc.VectorSubcoreMesh(**_MESH)

    @functools.partial(
        pl.kernel,
        out_type=jax.ShapeDtypeStruct((4, NP, 128), jnp.float32),
        mesh=mesh,
        scratch_types=[
            pltpu.VMEM((BA,), jnp.int32),
            pltpu.VMEM((BA,), jnp.int32),
            pltpu.VMEM((BA, 128), jnp.float32),
            pltpu.VMEM_SHARED((NP, 128), jnp.float32),
            pltpu.SemaphoreType.DMA,
        ],
    )
    def k(xw_h, s_h, d_h, z_h, o_h, idxv, didxv, gbuf, acc, sem):
        cid = lax.axis_index("c")
        sid = lax.axis_index("s")
        for kk in range(2):
            chunk = cid * 2 + kk
            pltpu.sync_copy(z_h.at[pl.ds(sid * RPT, RPT)],
                            acc.at[pl.ds(sid * RPT, RPT)])
            plsc.subcore_barrier()

            def blk(b, _):
                e0 = sid * 10000 + b * BA
                c1 = pltpu.async_copy(s_h.at[pl.ds(e0, BA)], idxv, sem)
                c2 = pltpu.async_copy(d_h.at[pl.ds(e0, BA)], didxv, sem)
                c1.wait()
                c2.wait()
                pltpu.sync_copy(xw_h.at[chunk].at[idxv], gbuf)
                pltpu.sync_copy(gbuf, acc.at[didxv], add=True)
                return 0

            lax.fori_loop(0, EE // 16 // BA, blk, 0)
            plsc.subcore_barrier()
            pltpu.sync_copy(acc.at[pl.ds(sid * RPT, RPT)],
                            o_h.at[chunk].at[pl.ds(sid * RPT, RPT)])
            plsc.subcore_barrier()

    return k(xwc, srci, dsti, zrows)


def kernel(x, edge_index, edge_attr, params):
    p = params
    n, in_f = x.shape
    src = edge_index[0]
    dst = edge_index[1]

    xp = jnp.zeros((NP, 128), jnp.float32).at[:n, :in_f].set(x)
    inwT = jnp.zeros((128, DD), jnp.float32).at[:in_f, :].set(p['in_w'].T)
    inb4 = p['in_b'].reshape(4, 1, 128)
    ea8 = jnp.zeros((EE, 8), jnp.float32).at[:, :4].set(edge_attr)
    gewT = jnp.zeros((8, DD), jnp.float32).at[:4, :].set(p['gine_edge_w'].T)
    geb4 = p['gine_edge_b'].reshape(4, 1, 128)
    gwT = jnp.zeros((8, HH * DD), jnp.float32).at[:4, :].set(p['gat_we'].T)
    zb16 = jnp.zeros((16, 1, 128), jnp.float32)
    att16 = p['gat_att'].reshape(16, 128)
    zrows = jnp.zeros((NP, 128), jnp.float32)
    zrows16 = jnp.zeros((NP, 16), jnp.float32)

    h0c = _tc_inproj(xp, inwT, inb4)

    elinc = _tc_edge_mm(ea8, gewT, geb4, 4)
    agg4 = _sc_gine_agg(h0c, elinc, src, dst, zrows)
    h1c = _tc_gine_mlp(h0c, agg4, p['gine_w1'].T, p['gine_b1'][None],
                       p['gine_w2'].T, p['gine_b2'][None],
                       p['ln0_g'][None], p['ln0_b'][None])

    xlg, xrg = _tc_gat_lin(h1c, p['gat_wl'].T, p['gat_wr'].T,
                           p['gat_bl'].reshape(16, 1, 128),
                           p['gat_br'].reshape(16, 1, 128))
    eeg = _tc_edge_mm(ea8, gwT, zb16, 16)
    logit16 = _sc_gat_logits(xlg, xrg, eeg, att16, src, dst)
    cmax = _tc_logit_max(logit16)
    ex16, exh = _tc_exp(logit16, cmax)
    denom2 = _sc_denom(ex16, dst, zrows16)
    rcph, dinv16 = _tc_denom_finish(denom2)
    gat16 = _sc_gat_out(xlg, exh, src, dst, zrows)
    h2c, xw2c = _tc_gat_finish(gat16, rcph, dinv16, h1c, p['gat_bias'][None],
                               p['ln1_g'][None], p['ln1_b'][None],
                               p['gcn2_w'].T)

    agg2 = _sc_gcn_agg(xw2c, src, dst, zrows)
    h3c, xw3c = _tc_gcn_finish(agg2, xw2c, dinv16, h2c, p['gcn2_b'][None],
                               p['ln2_g'][None], p['ln2_b'][None],
                               p['gcn3_w'].T, True)
    agg3 = _sc_gcn_agg(xw3c, src, dst, zrows)
    h4c, _ = _tc_gcn_finish(agg3, xw3c, dinv16, h3c, p['gcn3_b'][None],
                            p['ln3_g'][None], p['ln3_b'][None],
                            p['gcn3_w'].T, False)

    return jnp.concatenate([h4c[c][:n] for c in range(4)], axis=-1)


# speedup vs baseline: 4.6623x; 1.1759x over previous
import functools
import jax
import jax.numpy as jnp
from jax import lax
from jax.experimental import pallas as pl
from jax.experimental.pallas import tpu as pltpu, tpu_sc as plsc

NN = 10000
NP = 10240
EE = 160000
DD = 512
HH = 4
BM = 512
BE = 2000
BA = 80
BB = 40
RPT = NP // 16

_MESH = dict(core_axis_name="c", subcore_axis_name="s")


def _ln(z, g, b):
    mu = jnp.mean(z, axis=-1, keepdims=True)
    var = jnp.mean((z - mu) ** 2, axis=-1, keepdims=True)
    return (z - mu) * jax.lax.rsqrt(var + 1e-5) * g + b


def _cat4(ref):
    return jnp.concatenate([ref[0], ref[1], ref[2], ref[3]], axis=-1)


def _tc_inproj(xp, wT, b):
    def body(x_ref, w_ref, b_ref, o_ref):
        z = jnp.maximum(
            jnp.dot(x_ref[...], w_ref[...],
                    preferred_element_type=jnp.float32) + b_ref[0], 0.0)
        o_ref[...] = z[None]

    return pl.pallas_call(
        body,
        grid=(NP // BM, 4),
        in_specs=[
            pl.BlockSpec((BM, 128), lambda i, j: (i, 0)),
            pl.BlockSpec((128, 128), lambda i, j: (0, j)),
            pl.BlockSpec((1, 1, 128), lambda i, j: (j, 0, 0)),
        ],
        out_specs=pl.BlockSpec((1, BM, 128), lambda i, j: (j, i, 0)),
        out_shape=jax.ShapeDtypeStruct((4, NP, 128), jnp.float32),
    )(xp, wT, b)


def _tc_edge_mm(ea8, wT, b, ng):
    def body(a_ref, w_ref, b_ref, o_ref):
        z = jnp.dot(a_ref[...], w_ref[...],
                    preferred_element_type=jnp.float32) + b_ref[0]
        o_ref[...] = z[None]

    return pl.pallas_call(
        body,
        grid=(EE // BE, ng),
        in_specs=[
            pl.BlockSpec((BE, 8), lambda i, j: (i, 0)),
            pl.BlockSpec((8, 128), lambda i, j: (0, j)),
            pl.BlockSpec((1, 1, 128), lambda i, j: (j, 0, 0)),
        ],
        out_specs=pl.BlockSpec((1, BE, 128), lambda i, j: (j, i, 0)),
        out_shape=jax.ShapeDtypeStruct((ng, EE, 128), jnp.float32),
    )(ea8, wT, b)


def _tc_gine_mlp(h0c, agg4, w1T, b1, w2T, b2, lng, lnb):
    def body(h_ref, a_ref, w1_ref, b1_ref, w2_ref, b2_ref, g_ref, be_ref,
             o_ref):
        h0 = _cat4(h_ref)
        z = h0 + _cat4(a_ref)
        z = jnp.maximum(
            jnp.dot(z, w1_ref[...], preferred_element_type=jnp.float32)
            + b1_ref[...], 0.0)
        z = jnp.dot(z, w2_ref[...],
                    preferred_element_type=jnp.float32) + b2_ref[...]
        z = _ln(jnp.maximum(z, 0.0) + h0, g_ref[...], be_ref[...])
        for c in range(4):
            o_ref[c] = z[:, c * 128:(c + 1) * 128]

    return pl.pallas_call(
        body,
        grid=(NP // BM,),
        in_specs=[
            pl.BlockSpec((4, BM, 128), lambda i: (0, i, 0)),
            pl.BlockSpec((4, BM, 128), lambda i: (0, i, 0)),
            pl.BlockSpec((DD, 2 * DD), lambda i: (0, 0)),
            pl.BlockSpec((1, 2 * DD), lambda i: (0, 0)),
            pl.BlockSpec((2 * DD, DD), lambda i: (0, 0)),
            pl.BlockSpec((1, DD), lambda i: (0, 0)),
            pl.BlockSpec((1, DD), lambda i: (0, 0)),
            pl.BlockSpec((1, DD), lambda i: (0, 0)),
        ],
        out_specs=pl.BlockSpec((4, BM, 128), lambda i: (0, i, 0)),
        out_shape=jax.ShapeDtypeStruct((4, NP, 128), jnp.float32),
    )(h0c, agg4, w1T, b1, w2T, b2, lng, lnb)


def _tc_gat_lin(h1c, wlT, wrT, bl, br):
    def body(h_ref, wl_ref, wr_ref, bl_ref, br_ref, ol_ref, or_ref):
        h1 = _cat4(h_ref)
        ol_ref[...] = (jnp.dot(h1, wl_ref[...],
                               preferred_element_type=jnp.float32)
                       + bl_ref[0])[None]
        or_ref[...] = (jnp.dot(h1, wr_ref[...],
                               preferred_element_type=jnp.float32)
                       + br_ref[0])[None]

    return pl.pallas_call(
        body,
        grid=(NP // BM, 16),
        in_specs=[
            pl.BlockSpec((4, BM, 128), lambda i, j: (0, i, 0)),
            pl.BlockSpec((DD, 128), lambda i, j: (0, j)),
            pl.BlockSpec((DD, 128), lambda i, j: (0, j)),
            pl.BlockSpec((1, 1, 128), lambda i, j: (j, 0, 0)),
            pl.BlockSpec((1, 1, 128), lambda i, j: (j, 0, 0)),
        ],
        out_specs=[
            pl.BlockSpec((1, BM, 128), lambda i, j: (j, i, 0)),
            pl.BlockSpec((1, BM, 128), lambda i, j: (j, i, 0)),
        ],
        out_shape=[
            jax.ShapeDtypeStruct((16, NP, 128), jnp.float32),
            jax.ShapeDtypeStruct((16, NP, 128), jnp.float32),
        ],
    )(h1c, wlT, wrT, bl, br)


def _tc_logit_max(logit16):
    def body(l_ref, o_ref):
        i = pl.program_id(0)
        lane = lax.broadcasted_iota(jnp.int32, (BE, 16), 1)
        m = jnp.max(jnp.where(lane < HH, l_ref[...], -3e38))

        @pl.when(i == 0)
        def _():
            o_ref[0, 0] = m

        @pl.when(i > 0)
        def _():
            o_ref[0, 0] = jnp.maximum(o_ref[0, 0], m)

    return pl.pallas_call(
        body,
        grid=(EE // BE,),
        in_specs=[pl.BlockSpec((BE, 16), lambda i: (i, 0))],
        out_specs=pl.BlockSpec(memory_space=pltpu.SMEM),
        out_shape=jax.ShapeDtypeStruct((1, 1), jnp.float32),
    )(logit16)


def _tc_exp(logit16, cmax):
    def body(l_ref, c_ref, oe_ref, oh_ref):
        c = c_ref[0, 0]
        lane = lax.broadcasted_iota(jnp.int32, (BE, 16), 1)
        ex = jnp.exp(l_ref[...] - c)
        oe_ref[...] = jnp.where(lane < HH, ex,
                                jnp.where(lane == HH, 1.0, 0.0))
        for h in range(HH):
            oh_ref[h] = jnp.broadcast_to(ex[:, h:h + 1], (BE, 16))

    return pl.pallas_call(
        body,
        grid=(EE // BE,),
        in_specs=[
            pl.BlockSpec((BE, 16), lambda i: (i, 0)),
            pl.BlockSpec(memory_space=pltpu.SMEM),
        ],
        out_specs=[
            pl.BlockSpec((BE, 16), lambda i: (i, 0)),
            pl.BlockSpec((4, BE, 16), lambda i: (0, i, 0)),
        ],
        out_shape=[
            jax.ShapeDtypeStruct((EE, 16), jnp.float32),
            jax.ShapeDtypeStruct((HH, EE, 16), jnp.float32),
        ],
    )(logit16, cmax)


def _tc_denom_finish(denom2):
    def body(d_ref, rcp_ref, dv_ref):
        d = d_ref[0] + d_ref[1]
        for h in range(HH):
            rcp_ref[h] = jnp.broadcast_to(
                1.0 / (d[:, h:h + 1] + 1e-16), (BM, 16))
        deg = d[:, HH:HH + 1] + 1.0
        dv_ref[...] = jnp.broadcast_to(jax.lax.rsqrt(deg), (BM, 16))

    return pl.pallas_call(
        body,
        grid=(NP // BM,),
        in_specs=[pl.BlockSpec((2, BM, 16), lambda i: (0, i, 0))],
        out_specs=[
            pl.BlockSpec((4, BM, 16), lambda i: (0, i, 0)),
            pl.BlockSpec((BM, 16), lambda i: (i, 0)),
        ],
        out_shape=[
            jax.ShapeDtypeStruct((HH, NP, 16), jnp.float32),
            jax.ShapeDtypeStruct((NP, 16), jnp.float32),
        ],
    )(denom2)


def _tc_gat_finish(gat16, rcph, dinv16, h1c, bias, lng, lnb, wT):
    def body(g16_ref, rcp_ref, dv_ref, h_ref, b_ref, g_ref, be_ref, w_ref,
             oh_ref, ox_ref):
        cs = []
        for c in range(4):
            s = (g16_ref[c] * rcp_ref[0][:, 0:1]
                 + g16_ref[4 + c] * rcp_ref[1][:, 0:1]
                 + g16_ref[8 + c] * rcp_ref[2][:, 0:1]
                 + g16_ref[12 + c] * rcp_ref[3][:, 0:1])
            cs.append(0.25 * s)
        z = jnp.concatenate(cs, axis=-1) + b_ref[...]
        res = _cat4(h_ref)
        h2 = _ln(jnp.maximum(z, 0.0) + res, g_ref[...], be_ref[...])
        xw = dv_ref[:, 0:1] * jnp.dot(h2, w_ref[...],
                                      preferred_element_type=jnp.float32)
        for c in range(4):
            oh_ref[c] = h2[:, c * 128:(c + 1) * 128]
            ox_ref[c] = xw[:, c * 128:(c + 1) * 128]

    return pl.pallas_call(
        body,
        grid=(NP // BM,),
        in_specs=[
            pl.BlockSpec((16, BM, 128), lambda i: (0, i, 0)),
            pl.BlockSpec((4, BM, 16), lambda i: (0, i, 0)),
            pl.BlockSpec((BM, 16), lambda i: (i, 0)),
            pl.BlockSpec((4, BM, 128), lambda i: (0, i, 0)),
            pl.BlockSpec((1, DD), lambda i: (0, 0)),
            pl.BlockSpec((1, DD), lambda i: (0, 0)),
            pl.BlockSpec((1, DD), lambda i: (0, 0)),
            pl.BlockSpec((DD, DD), lambda i: (0, 0)),
        ],
        out_specs=[
            pl.BlockSpec((4, BM, 128), lambda i: (0, i, 0)),
            pl.BlockSpec((4, BM, 128), lambda i: (0, i, 0)),
        ],
        out_shape=[
            jax.ShapeDtypeStruct((4, NP, 128), jnp.float32),
            jax.ShapeDtypeStruct((4, NP, 128), jnp.float32),
        ],
    )(gat16, rcph, dinv16, h1c, bias, lng, lnb, wT)


def _tc_gcn_finish(agg4, xwc, dinv16, hres, bias, lng, lnb, wT, make_xw):
    def body(a_ref, x_ref, d_ref, h_ref, b_ref, g_ref, be_ref, w_ref,
             oh_ref, ox_ref):
        d1 = d_ref[:, 0:1]
        z = d1 * (_cat4(a_ref) + _cat4(x_ref)) + b_ref[...]
        res = _cat4(h_ref)
        hn = _ln(jnp.maximum(z, 0.0) + res, g_ref[...], be_ref[...])
        for c in range(4):
            oh_ref[c] = hn[:, c * 128:(c + 1) * 128]
        if make_xw:
            xw = d1 * jnp.dot(hn, w_ref[...],
                              preferred_element_type=jnp.float32)
            for c in range(4):
                ox_ref[c] = xw[:, c * 128:(c + 1) * 128]
        else:
            ox_ref[0] = jnp.zeros((BM, 128), jnp.float32)

    return pl.pallas_call(
        body,
        grid=(NP // BM,),
        in_specs=[
            pl.BlockSpec((4, BM, 128), lambda i: (0, i, 0)),
            pl.BlockSpec((4, BM, 128), lambda i: (0, i, 0)),
            pl.BlockSpec((BM, 16), lambda i: (i, 0)),
            pl.BlockSpec((4, BM, 128), lambda i: (0, i, 0)),
            pl.BlockSpec((1, DD), lambda i: (0, 0)),
            pl.BlockSpec((1, DD), lambda i: (0, 0)),
            pl.BlockSpec((1, DD), lambda i: (0, 0)),
            pl.BlockSpec((DD, DD), lambda i: (0, 0)),
        ],
        out_specs=[
            pl.BlockSpec((4, BM, 128), lambda i: (0, i, 0)),
            pl.BlockSpec((4, BM, 128) if make_xw else (1, BM, 128),
                         (lambda i: (0, i, 0)) if make_xw
                         else (lambda i: (0, i, 0))),
        ],
        out_shape=[
            jax.ShapeDtypeStruct((4, NP, 128), jnp.float32),
            jax.ShapeDtypeStruct((4, NP, 128) if make_xw else (1, NP, 128),
                                 jnp.float32),
        ],
    )(agg4, xwc, dinv16, hres, bias, lng, lnb, wT)


def _pipe2(nblk, issue_idx, wait_idx, issue_data, wait_data, work):
    issue_idx(0, 0)
    wait_idx(0, 0)
    issue_data(0, 0)
    issue_idx(1, 1)

    def step(b, s):
        wait_idx(b + 1, 1 - s)
        issue_data(b + 1, 1 - s)
        wait_data(b, s)
        work(b, s)
        issue_idx(b + 2, s)

    def body(g, _):
        step(2 * g, 0)
        step(2 * g + 1, 1)
        return 0

    lax.fori_loop(0, (nblk - 1) // 2, body, 0)
    wait_idx(nblk, 1)
    wait_data(nblk - 1, 0)
    work(nblk - 1, 0)

def _sc_gine_agg(h0c, elinc, srci, dsti, zrows):
    mesh = plsc.VectorSubcoreMesh(**_MESH)

    @functools.partial(
        pl.kernel,
        out_type=jax.ShapeDtypeStruct((4, NP, 128), jnp.float32),
        mesh=mesh,
        scratch_types=[
            [pltpu.VMEM((BA,), jnp.int32) for _ in range(2)],
            [pltpu.VMEM((BA,), jnp.int32) for _ in range(2)],
            [pltpu.VMEM((BA, 128), jnp.float32) for _ in range(2)],
            [pltpu.VMEM((BA, 128), jnp.float32) for _ in range(2)],
            pltpu.VMEM_SHARED((NP, 128), jnp.float32),
            [pltpu.SemaphoreType.DMA for _ in range(2)],
            [pltpu.SemaphoreType.DMA for _ in range(2)],
        ],
    )
    def k(h_h, e_h, s_h, d_h, z_h, o_h, idxv, didxv, gbuf, ebuf, acc, semi,
          semd):
        cid = lax.axis_index("c")
        sid = lax.axis_index("s")
        nblk = EE // 16 // BA
        for kk in range(2):
            chunk = cid * 2 + kk
            pltpu.sync_copy(z_h.at[pl.ds(sid * RPT, RPT)],
                            acc.at[pl.ds(sid * RPT, RPT)])
            plsc.subcore_barrier()

            def e_of(b):
                return sid * 10000 + jnp.minimum(b, nblk - 1) * BA

            def issue_idx(b, s):
                e0 = e_of(b)
                pltpu.async_copy(s_h.at[pl.ds(e0, BA)], idxv[s], semi[s])
                pltpu.async_copy(d_h.at[pl.ds(e0, BA)], didxv[s], semi[s])

            def wait_idx(b, s):
                e0 = e_of(b)
                pltpu.make_async_copy(s_h.at[pl.ds(e0, BA)], idxv[s],
                                      semi[s]).wait()
                pltpu.make_async_copy(d_h.at[pl.ds(e0, BA)], didxv[s],
                                      semi[s]).wait()

            def issue_data(b, s):
                e0 = e_of(b)
                pltpu.async_copy(e_h.at[chunk].at[pl.ds(e0, BA)], ebuf[s],
                                 semd[s])
                pltpu.async_copy(h_h.at[chunk].at[idxv[s]], gbuf[s],
                                 semd[s])

            def wait_data(b, s):
                e0 = e_of(b)
                pltpu.make_async_copy(e_h.at[chunk].at[pl.ds(e0, BA)],
                                      ebuf[s], semd[s]).wait()
                pltpu.make_async_copy(h_h.at[chunk].at[idxv[s]], gbuf[s],
                                      semd[s]).wait()

            def work(b, s):
                def rbody(r, _):
                    for v in range(8):
                        sl = pl.ds(v * 16, 16)
                        gbuf[s][r, sl] = jnp.maximum(
                            gbuf[s][r, sl] + ebuf[s][r, sl], 0.0)
                    return 0

                lax.fori_loop(0, BA, rbody, 0)
                pltpu.sync_copy(gbuf[s], acc.at[didxv[s]], add=True)

            _pipe2(nblk, issue_idx, wait_idx, issue_data, wait_data, work)
            plsc.subcore_barrier()
            pltpu.sync_copy(acc.at[pl.ds(sid * RPT, RPT)],
                            o_h.at[chunk].at[pl.ds(sid * RPT, RPT)])
            plsc.subcore_barrier()

    return k(h0c, elinc, srci, dsti, zrows)


def _sc_gat_logits(xlg, xrg, eeg, att16, srci, dsti):
    mesh = plsc.VectorSubcoreMesh(**_MESH)
    nb = EE // 32 // BB

    @functools.partial(
        pl.kernel,
        out_type=jax.ShapeDtypeStruct((EE, 16), jnp.float32),
        mesh=mesh,
        scratch_types=[
            pltpu.VMEM((BB,), jnp.int32),
            pltpu.VMEM((BB,), jnp.int32),
            [pltpu.VMEM((BB, 128), jnp.float32) for _ in range(4)],
            [pltpu.VMEM((BB, 128), jnp.float32) for _ in range(4)],
            [pltpu.VMEM((BB, 128), jnp.float32) for _ in range(4)],
            pltpu.VMEM((16, 128), jnp.float32),
            pltpu.VMEM((BB, 16), jnp.float32),
            pltpu.SemaphoreType.DMA,
        ],
    )
    def k(xl_h, xr_h, ee_h, att_h, s_h, d_h, o_h, idxv, didxv, xlb, xrb,
          eeb, attb, lbuf, sem):
        cid = lax.axis_index("c")
        sid = lax.axis_index("s")
        wid = cid * 16 + sid
        pltpu.sync_copy(att_h, attb)
        lanes = lax.iota(jnp.int32, 16)

        def blk(b, _):
            e0 = wid * (EE // 32) + b * BB
            c1 = pltpu.async_copy(s_h.at[pl.ds(e0, BB)], idxv, sem)
            c2 = pltpu.async_copy(d_h.at[pl.ds(e0, BB)], didxv, sem)
            c1.wait()
            c2.wait()
            for h in range(HH):
                cps = []
                for c in range(4):
                    g = h * 4 + c
                    cps.append(pltpu.async_copy(
                        xl_h.at[g].at[idxv], xlb[c], sem))
                    cps.append(pltpu.async_copy(
                        xr_h.at[g].at[didxv], xrb[c], sem))
                    cps.append(pltpu.async_copy(
                        ee_h.at[g].at[pl.ds(e0, BB)], eeb[c], sem))
                for cp in cps:
                    cp.wait()

                def rbody(r, _):
                    accv = jnp.zeros((16,), jnp.float32)
                    for c in range(4):
                        for v in range(8):
                            sl = pl.ds(v * 16, 16)
                            s = xlb[c][r, sl] + xrb[c][r, sl] + eeb[c][r, sl]
                            m = jnp.maximum(s, 0.0) + 0.2 * jnp.minimum(
                                s, 0.0)
                            accv = accv + m * attb[h * 4 + c, sl]
                    for sh in (8, 4, 2, 1):
                        accv = accv + accv.at[lanes ^ sh].get(
                            mode="promise_in_bounds")
                    if h == 0:
                        row = jnp.where(lanes == 0, accv, 0.0)
                    else:
                        row = jnp.where(lanes == h, accv,
                                        lbuf[r, pl.ds(0, 16)])
                    lbuf[r, pl.ds(0, 16)] = row
                    return 0

                lax.fori_loop(0, BB, rbody, 0)
            pltpu.sync_copy(lbuf, o_h.at[pl.ds(e0, BB)])
            return 0

        lax.fori_loop(0, nb, blk, 0)

    return k(xlg, xrg, eeg, att16, srci, dsti)


def _sc_denom(ex16, dsti, zrows16):
    mesh = plsc.VectorSubcoreMesh(**_MESH)
    nb = EE // 32 // BB

    @functools.partial(
        pl.kernel,
        out_type=jax.ShapeDtypeStruct((2, NP, 16), jnp.float32),
        mesh=mesh,
        scratch_types=[
            pltpu.VMEM((BB,), jnp.int32),
            pltpu.VMEM((BB, 16), jnp.float32),
            pltpu.VMEM_SHARED((NP, 16), jnp.float32),
            pltpu.SemaphoreType.DMA,
        ],
    )
    def k(ex_h, d_h, z_h, o_h, didxv, exb, acc, sem):
        cid = lax.axis_index("c")
        sid = lax.axis_index("s")
        pltpu.sync_copy(z_h.at[pl.ds(sid * RPT, RPT)],
                        acc.at[pl.ds(sid * RPT, RPT)])
        plsc.subcore_barrier()

        def blk(b, _):
            e0 = (cid * 16 + sid) * (EE // 32) + b * BB
            c1 = pltpu.async_copy(d_h.at[pl.ds(e0, BB)], didxv, sem)
            c2 = pltpu.async_copy(ex_h.at[pl.ds(e0, BB)], exb, sem)
            c1.wait()
            c2.wait()
            pltpu.sync_copy(exb, acc.at[didxv], add=True)
            return 0

        lax.fori_loop(0, nb, blk, 0)
        plsc.subcore_barrier()
        pltpu.sync_copy(acc.at[pl.ds(sid * RPT, RPT)],
                        o_h.at[cid].at[pl.ds(sid * RPT, RPT)])

    return k(ex16, dsti, zrows16)


def _sc_gat_out(xlg, exh, srci, dsti, zrows):
    mesh = plsc.VectorSubcoreMesh(**_MESH)

    @functools.partial(
        pl.kernel,
        out_type=jax.ShapeDtypeStruct((16, NP, 128), jnp.float32),
        mesh=mesh,
        scratch_types=[
            [pltpu.VMEM((BA,), jnp.int32) for _ in range(2)],
            [pltpu.VMEM((BA,), jnp.int32) for _ in range(2)],
            [pltpu.VMEM((BA, 128), jnp.float32) for _ in range(2)],
            [pltpu.VMEM((BA, 16), jnp.float32) for _ in range(2)],
            pltpu.VMEM_SHARED((NP, 128), jnp.float32),
            [pltpu.SemaphoreType.DMA for _ in range(2)],
            [pltpu.SemaphoreType.DMA for _ in range(2)],
        ],
    )
    def k(xl_h, ex_h, s_h, d_h, z_h, o_h, idxv, didxv, gbuf, exb, acc, semi,
          semd):
        cid = lax.axis_index("c")
        sid = lax.axis_index("s")
        nblk = EE // 16 // BA
        for g8 in range(8):
            g = cid * 8 + g8
            h = cid * 2 + (g8 // 4)
            pltpu.sync_copy(z_h.at[pl.ds(sid * RPT, RPT)],
                            acc.at[pl.ds(sid * RPT, RPT)])
            plsc.subcore_barrier()

            def e_of(b):
                return sid * 10000 + jnp.minimum(b, nblk - 1) * BA

            def issue_idx(b, s):
                e0 = e_of(b)
                pltpu.async_copy(s_h.at[pl.ds(e0, BA)], idxv[s], semi[s])
                pltpu.async_copy(d_h.at[pl.ds(e0, BA)], didxv[s], semi[s])

            def wait_idx(b, s):
                e0 = e_of(b)
                pltpu.make_async_copy(s_h.at[pl.ds(e0, BA)], idxv[s],
                                      semi[s]).wait()
                pltpu.make_async_copy(d_h.at[pl.ds(e0, BA)], didxv[s],
                                      semi[s]).wait()

            def issue_data(b, s):
                e0 = e_of(b)
                pltpu.async_copy(ex_h.at[h].at[pl.ds(e0, BA)], exb[s],
                                 semd[s])
                pltpu.async_copy(xl_h.at[g].at[idxv[s]], gbuf[s], semd[s])

            def wait_data(b, s):
                e0 = e_of(b)
                pltpu.make_async_copy(ex_h.at[h].at[pl.ds(e0, BA)], exb[s],
                                      semd[s]).wait()
                pltpu.make_async_copy(xl_h.at[g].at[idxv[s]], gbuf[s],
                                      semd[s]).wait()

            def work(b, s):
                def rbody(r, _):
                    av = exb[s][r, pl.ds(0, 16)]
                    for v in range(8):
                        sl = pl.ds(v * 16, 16)
                        gbuf[s][r, sl] = gbuf[s][r, sl] * av
                    return 0

                lax.fori_loop(0, BA, rbody, 0)
                pltpu.sync_copy(gbuf[s], acc.at[didxv[s]], add=True)

            _pipe2(nblk, issue_idx, wait_idx, issue_data, wait_data, work)
            plsc.subcore_barrier()
            pltpu.sync_copy(acc.at[pl.ds(sid * RPT, RPT)],
                            o_h.at[g].at[pl.ds(sid * RPT, RPT)])
            plsc.subcore_barrier()

    return k(xlg, exh, srci, dsti, zrows)


def _sc_gcn_agg(xwc, srci, dsti, zrows):
    mesh = plsc.VectorSubcoreMesh(**_MESH)

    @functools.partial(
        pl.kernel,
        out_type=jax.ShapeDtypeStruct((4, NP, 128), jnp.float32),
        mesh=mesh,
        scratch_types=[
            [pltpu.VMEM((BA,), jnp.int32) for _ in range(2)],
            [pltpu.VMEM((BA,), jnp.int32) for _ in range(2)],
            [pltpu.VMEM((BA, 128), jnp.float32) for _ in range(2)],
            pltpu.VMEM_SHARED((NP, 128), jnp.float32),
            [pltpu.SemaphoreType.DMA for _ in range(2)],
            [pltpu.SemaphoreType.DMA for _ in range(2)],
        ],
    )
    def k(xw_h, s_h, d_h, z_h, o_h, idxv, didxv, gbuf, acc, semi, semd):
        cid = lax.axis_index("c")
        sid = lax.axis_index("s")
        nblk = EE // 16 // BA
        for kk in range(2):
            chunk = cid * 2 + kk
            pltpu.sync_copy(z_h.at[pl.ds(sid * RPT, RPT)],
                            acc.at[pl.ds(sid * RPT, RPT)])
            plsc.subcore_barrier()

            def e_of(b):
                return sid * 10000 + jnp.minimum(b, nblk - 1) * BA

            def issue_idx(b, s):
                e0 = e_of(b)
                pltpu.async_copy(s_h.at[pl.ds(e0, BA)], idxv[s], semi[s])
                pltpu.async_copy(d_h.at[pl.ds(e0, BA)], didxv[s], semi[s])

            def wait_idx(b, s):
                e0 = e_of(b)
                pltpu.make_async_copy(s_h.at[pl.ds(e0, BA)], idxv[s],
                                      semi[s]).wait()
                pltpu.make_async_copy(d_h.at[pl.ds(e0, BA)], didxv[s],
                                      semi[s]).wait()

            def issue_data(b, s):
                pltpu.async_copy(xw_h.at[chunk].at[idxv[s]], gbuf[s],
                                 semd[s])

            def wait_data(b, s):
                pltpu.make_async_copy(xw_h.at[chunk].at[idxv[s]], gbuf[s],
                                      semd[s]).wait()

            def work(b, s):
                pltpu.sync_copy(gbuf[s], acc.at[didxv[s]], add=True)

            _pipe2(nblk, issue_idx, wait_idx, issue_data, wait_data, work)
            plsc.subcore_barrier()
            pltpu.sync_copy(acc.at[pl.ds(sid * RPT, RPT)],
                            o_h.at[chunk].at[pl.ds(sid * RPT, RPT)])
            plsc.subcore_barrier()

    return k(xwc, srci, dsti, zrows)


def kernel(x, edge_index, edge_attr, params):
    p = params
    n, in_f = x.shape
    src = edge_index[0]
    dst = edge_index[1]

    xp = jnp.zeros((NP, 128), jnp.float32).at[:n, :in_f].set(x)
    inwT = jnp.zeros((128, DD), jnp.float32).at[:in_f, :].set(p['in_w'].T)
    inb4 = p['in_b'].reshape(4, 1, 128)
    ea8 = jnp.zeros((EE, 8), jnp.float32).at[:, :4].set(edge_attr)
    gewT = jnp.zeros((8, DD), jnp.float32).at[:4, :].set(p['gine_edge_w'].T)
    geb4 = p['gine_edge_b'].reshape(4, 1, 128)
    gwT = jnp.zeros((8, HH * DD), jnp.float32).at[:4, :].set(p['gat_we'].T)
    zb16 = jnp.zeros((16, 1, 128), jnp.float32)
    att16 = p['gat_att'].reshape(16, 128)
    zrows = jnp.zeros((NP, 128), jnp.float32)
    zrows16 = jnp.zeros((NP, 16), jnp.float32)

    h0c = _tc_inproj(xp, inwT, inb4)

    elinc = _tc_edge_mm(ea8, gewT, geb4, 4)
    agg4 = _sc_gine_agg(h0c, elinc, src, dst, zrows)
    h1c = _tc_gine_mlp(h0c, agg4, p['gine_w1'].T, p['gine_b1'][None],
                       p['gine_w2'].T, p['gine_b2'][None],
                       p['ln0_g'][None], p['ln0_b'][None])

    xlg, xrg = _tc_gat_lin(h1c, p['gat_wl'].T, p['gat_wr'].T,
                           p['gat_bl'].reshape(16, 1, 128),
                           p['gat_br'].reshape(16, 1, 128))
    eeg = _tc_edge_mm(ea8, gwT, zb16, 16)
    logit16 = _sc_gat_logits(xlg, xrg, eeg, att16, src, dst)
    cmax = _tc_logit_max(logit16)
    ex16, exh = _tc_exp(logit16, cmax)
    denom2 = _sc_denom(ex16, dst, zrows16)
    rcph, dinv16 = _tc_denom_finish(denom2)
    gat16 = _sc_gat_out(xlg, exh, src, dst, zrows)
    h2c, xw2c = _tc_gat_finish(gat16, rcph, dinv16, h1c, p['gat_bias'][None],
                               p['ln1_g'][None], p['ln1_b'][None],
                               p['gcn2_w'].T)

    agg2 = _sc_gcn_agg(xw2c, src, dst, zrows)
    h3c, xw3c = _tc_gcn_finish(agg2, xw2c, dinv16, h2c, p['gcn2_b'][None],
                               p['ln2_g'][None], p['ln2_b'][None],
                               p['gcn3_w'].T, True)
    agg3 = _sc_gcn_agg(xw3c, src, dst, zrows)
    h4c, _ = _tc_gcn_finish(agg3, xw3c, dinv16, h3c, p['gcn3_b'][None],
                            p['ln3_g'][None], p['ln3_b'][None],
                            p['gcn3_w'].T, False)

    return jnp.concatenate([h4c[c][:n] for c in range(4)], axis=-1)


# speedup vs baseline: 5.0704x; 1.0875x over previous
import functools
import jax
import jax.numpy as jnp
from jax import lax
from jax.experimental import pallas as pl
from jax.experimental.pallas import tpu as pltpu, tpu_sc as plsc

NN = 10000
NP = 10240
EE = 160000
DD = 512
HH = 4
BM = 512
BE = 2000
BA = 80
BB = 40
RPT = NP // 16

_MESH = dict(core_axis_name="c", subcore_axis_name="s")


def _ln(z, g, b):
    mu = jnp.mean(z, axis=-1, keepdims=True)
    var = jnp.mean((z - mu) ** 2, axis=-1, keepdims=True)
    return (z - mu) * jax.lax.rsqrt(var + 1e-5) * g + b


def _cat4(ref):
    return jnp.concatenate([ref[0], ref[1], ref[2], ref[3]], axis=-1)


def _tc_inproj(xp, wT, b):
    def body(x_ref, w_ref, b_ref, o_ref):
        z = jnp.maximum(
            jnp.dot(x_ref[...], w_ref[...],
                    preferred_element_type=jnp.float32) + b_ref[0], 0.0)
        o_ref[...] = z[None]

    return pl.pallas_call(
        body,
        grid=(NP // BM, 4),
        in_specs=[
            pl.BlockSpec((BM, 128), lambda i, j: (i, 0)),
            pl.BlockSpec((128, 128), lambda i, j: (0, j)),
            pl.BlockSpec((1, 1, 128), lambda i, j: (j, 0, 0)),
        ],
        out_specs=pl.BlockSpec((1, BM, 128), lambda i, j: (j, i, 0)),
        out_shape=jax.ShapeDtypeStruct((4, NP, 128), jnp.float32),
    )(xp, wT, b)


def _tc_edge_mm(ea8, wT, b, ng):
    def body(a_ref, w_ref, b_ref, o_ref):
        z = jnp.dot(a_ref[...], w_ref[...],
                    preferred_element_type=jnp.float32) + b_ref[0]
        o_ref[...] = z[None]

    return pl.pallas_call(
        body,
        grid=(EE // BE, ng),
        in_specs=[
            pl.BlockSpec((BE, 8), lambda i, j: (i, 0)),
            pl.BlockSpec((8, 128), lambda i, j: (0, j)),
            pl.BlockSpec((1, 1, 128), lambda i, j: (j, 0, 0)),
        ],
        out_specs=pl.BlockSpec((1, BE, 128), lambda i, j: (j, i, 0)),
        out_shape=jax.ShapeDtypeStruct((ng, EE, 128), jnp.float32),
    )(ea8, wT, b)


def _tc_gine_mlp(h0c, agg4, w1T, b1, w2T, b2, lng, lnb):
    def body(h_ref, a_ref, w1_ref, b1_ref, w2_ref, b2_ref, g_ref, be_ref,
             o_ref):
        h0 = _cat4(h_ref)
        z = h0 + _cat4(a_ref)
        z = jnp.maximum(
            jnp.dot(z, w1_ref[...], preferred_element_type=jnp.float32)
            + b1_ref[...], 0.0)
        z = jnp.dot(z, w2_ref[...],
                    preferred_element_type=jnp.float32) + b2_ref[...]
        z = _ln(jnp.maximum(z, 0.0) + h0, g_ref[...], be_ref[...])
        for c in range(4):
            o_ref[c] = z[:, c * 128:(c + 1) * 128]

    return pl.pallas_call(
        body,
        grid=(NP // BM,),
        in_specs=[
            pl.BlockSpec((4, BM, 128), lambda i: (0, i, 0)),
            pl.BlockSpec((4, BM, 128), lambda i: (0, i, 0)),
            pl.BlockSpec((DD, 2 * DD), lambda i: (0, 0)),
            pl.BlockSpec((1, 2 * DD), lambda i: (0, 0)),
            pl.BlockSpec((2 * DD, DD), lambda i: (0, 0)),
            pl.BlockSpec((1, DD), lambda i: (0, 0)),
            pl.BlockSpec((1, DD), lambda i: (0, 0)),
            pl.BlockSpec((1, DD), lambda i: (0, 0)),
        ],
        out_specs=pl.BlockSpec((4, BM, 128), lambda i: (0, i, 0)),
        out_shape=jax.ShapeDtypeStruct((4, NP, 128), jnp.float32),
    )(h0c, agg4, w1T, b1, w2T, b2, lng, lnb)


def _tc_gat_lin(h1c, wlT, wrT, bl, br):
    def body(h_ref, wl_ref, wr_ref, bl_ref, br_ref, ol_ref, or_ref):
        h1 = _cat4(h_ref)
        ol_ref[...] = (jnp.dot(h1, wl_ref[...],
                               preferred_element_type=jnp.float32)
                       + bl_ref[0])[None]
        or_ref[...] = (jnp.dot(h1, wr_ref[...],
                               preferred_element_type=jnp.float32)
                       + br_ref[0])[None]

    return pl.pallas_call(
        body,
        grid=(NP // BM, 16),
        in_specs=[
            pl.BlockSpec((4, BM, 128), lambda i, j: (0, i, 0)),
            pl.BlockSpec((DD, 128), lambda i, j: (0, j)),
            pl.BlockSpec((DD, 128), lambda i, j: (0, j)),
            pl.BlockSpec((1, 1, 128), lambda i, j: (j, 0, 0)),
            pl.BlockSpec((1, 1, 128), lambda i, j: (j, 0, 0)),
        ],
        out_specs=[
            pl.BlockSpec((1, BM, 128), lambda i, j: (j, i, 0)),
            pl.BlockSpec((1, BM, 128), lambda i, j: (j, i, 0)),
        ],
        out_shape=[
            jax.ShapeDtypeStruct((16, NP, 128), jnp.float32),
            jax.ShapeDtypeStruct((16, NP, 128), jnp.float32),
        ],
    )(h1c, wlT, wrT, bl, br)


def _tc_logit_max(logit16):
    def body(l_ref, o_ref):
        i = pl.program_id(0)
        lane = lax.broadcasted_iota(jnp.int32, (BE, 16), 1)
        m = jnp.max(jnp.where(lane < HH, l_ref[...], -3e38))

        @pl.when(i == 0)
        def _():
            o_ref[0, 0] = m

        @pl.when(i > 0)
        def _():
            o_ref[0, 0] = jnp.maximum(o_ref[0, 0], m)

    return pl.pallas_call(
        body,
        grid=(EE // BE,),
        in_specs=[pl.BlockSpec((BE, 16), lambda i: (i, 0))],
        out_specs=pl.BlockSpec(memory_space=pltpu.SMEM),
        out_shape=jax.ShapeDtypeStruct((1, 1), jnp.float32),
    )(logit16)


def _tc_exp(logit16, cmax):
    def body(l_ref, c_ref, oe_ref, oh_ref):
        c = c_ref[0, 0]
        lane = lax.broadcasted_iota(jnp.int32, (BE, 16), 1)
        ex = jnp.exp(l_ref[...] - c)
        oe_ref[...] = jnp.where(lane < HH, ex,
                                jnp.where(lane == HH, 1.0, 0.0))
        for h in range(HH):
            oh_ref[h] = jnp.broadcast_to(ex[:, h:h + 1], (BE, 16))

    return pl.pallas_call(
        body,
        grid=(EE // BE,),
        in_specs=[
            pl.BlockSpec((BE, 16), lambda i: (i, 0)),
            pl.BlockSpec(memory_space=pltpu.SMEM),
        ],
        out_specs=[
            pl.BlockSpec((BE, 16), lambda i: (i, 0)),
            pl.BlockSpec((4, BE, 16), lambda i: (0, i, 0)),
        ],
        out_shape=[
            jax.ShapeDtypeStruct((EE, 16), jnp.float32),
            jax.ShapeDtypeStruct((HH, EE, 16), jnp.float32),
        ],
    )(logit16, cmax)


def _tc_denom_finish(denom2):
    def body(d_ref, rcp_ref, dv_ref):
        d = d_ref[0] + d_ref[1]
        for h in range(HH):
            rcp_ref[h] = jnp.broadcast_to(
                1.0 / (d[:, h:h + 1] + 1e-16), (BM, 16))
        deg = d[:, HH:HH + 1] + 1.0
        dv_ref[...] = jnp.broadcast_to(jax.lax.rsqrt(deg), (BM, 16))

    return pl.pallas_call(
        body,
        grid=(NP // BM,),
        in_specs=[pl.BlockSpec((2, BM, 16), lambda i: (0, i, 0))],
        out_specs=[
            pl.BlockSpec((4, BM, 16), lambda i: (0, i, 0)),
            pl.BlockSpec((BM, 16), lambda i: (i, 0)),
        ],
        out_shape=[
            jax.ShapeDtypeStruct((HH, NP, 16), jnp.float32),
            jax.ShapeDtypeStruct((NP, 16), jnp.float32),
        ],
    )(denom2)


def _tc_gat_finish(gat16, rcph, dinv16, h1c, bias, lng, lnb, wT):
    def body(g16_ref, rcp_ref, dv_ref, h_ref, b_ref, g_ref, be_ref, w_ref,
             oh_ref, ox_ref):
        cs = []
        for c in range(4):
            s = (g16_ref[c] * rcp_ref[0][:, 0:1]
                 + g16_ref[4 + c] * rcp_ref[1][:, 0:1]
                 + g16_ref[8 + c] * rcp_ref[2][:, 0:1]
                 + g16_ref[12 + c] * rcp_ref[3][:, 0:1])
            cs.append(0.25 * s)
        z = jnp.concatenate(cs, axis=-1) + b_ref[...]
        res = _cat4(h_ref)
        h2 = _ln(jnp.maximum(z, 0.0) + res, g_ref[...], be_ref[...])
        xw = dv_ref[:, 0:1] * jnp.dot(h2, w_ref[...],
                                      preferred_element_type=jnp.float32)
        for c in range(4):
            oh_ref[c] = h2[:, c * 128:(c + 1) * 128]
            ox_ref[c] = xw[:, c * 128:(c + 1) * 128]

    return pl.pallas_call(
        body,
        grid=(NP // BM,),
        in_specs=[
            pl.BlockSpec((16, BM, 128), lambda i: (0, i, 0)),
            pl.BlockSpec((4, BM, 16), lambda i: (0, i, 0)),
            pl.BlockSpec((BM, 16), lambda i: (i, 0)),
            pl.BlockSpec((4, BM, 128), lambda i: (0, i, 0)),
            pl.BlockSpec((1, DD), lambda i: (0, 0)),
            pl.BlockSpec((1, DD), lambda i: (0, 0)),
            pl.BlockSpec((1, DD), lambda i: (0, 0)),
            pl.BlockSpec((DD, DD), lambda i: (0, 0)),
        ],
        out_specs=[
            pl.BlockSpec((4, BM, 128), lambda i: (0, i, 0)),
            pl.BlockSpec((4, BM, 128), lambda i: (0, i, 0)),
        ],
        out_shape=[
            jax.ShapeDtypeStruct((4, NP, 128), jnp.float32),
            jax.ShapeDtypeStruct((4, NP, 128), jnp.float32),
        ],
    )(gat16, rcph, dinv16, h1c, bias, lng, lnb, wT)


def _tc_gcn_finish(agg4, xwc, dinv16, hres, bias, lng, lnb, wT, make_xw):
    def body(a_ref, x_ref, d_ref, h_ref, b_ref, g_ref, be_ref, w_ref,
             oh_ref, ox_ref):
        d1 = d_ref[:, 0:1]
        z = d1 * (_cat4(a_ref) + _cat4(x_ref)) + b_ref[...]
        res = _cat4(h_ref)
        hn = _ln(jnp.maximum(z, 0.0) + res, g_ref[...], be_ref[...])
        for c in range(4):
            oh_ref[c] = hn[:, c * 128:(c + 1) * 128]
        if make_xw:
            xw = d1 * jnp.dot(hn, w_ref[...],
                              preferred_element_type=jnp.float32)
            for c in range(4):
                ox_ref[c] = xw[:, c * 128:(c + 1) * 128]
        else:
            ox_ref[0] = jnp.zeros((BM, 128), jnp.float32)

    return pl.pallas_call(
        body,
        grid=(NP // BM,),
        in_specs=[
            pl.BlockSpec((4, BM, 128), lambda i: (0, i, 0)),
            pl.BlockSpec((4, BM, 128), lambda i: (0, i, 0)),
            pl.BlockSpec((BM, 16), lambda i: (i, 0)),
            pl.BlockSpec((4, BM, 128), lambda i: (0, i, 0)),
            pl.BlockSpec((1, DD), lambda i: (0, 0)),
            pl.BlockSpec((1, DD), lambda i: (0, 0)),
            pl.BlockSpec((1, DD), lambda i: (0, 0)),
            pl.BlockSpec((DD, DD), lambda i: (0, 0)),
        ],
        out_specs=[
            pl.BlockSpec((4, BM, 128), lambda i: (0, i, 0)),
            pl.BlockSpec((4, BM, 128) if make_xw else (1, BM, 128),
                         (lambda i: (0, i, 0)) if make_xw
                         else (lambda i: (0, i, 0))),
        ],
        out_shape=[
            jax.ShapeDtypeStruct((4, NP, 128), jnp.float32),
            jax.ShapeDtypeStruct((4, NP, 128) if make_xw else (1, NP, 128),
                                 jnp.float32),
        ],
    )(agg4, xwc, dinv16, hres, bias, lng, lnb, wT)


def _pipe2(nblk, issue_idx, wait_idx, issue_data, wait_data, work):
    issue_idx(0, 0)
    wait_idx(0, 0)
    issue_data(0, 0)
    issue_idx(1, 1)

    def step(b, s):
        wait_idx(b + 1, 1 - s)
        issue_data(b + 1, 1 - s)
        wait_data(b, s)
        work(b, s)
        issue_idx(b + 2, s)

    def body(g, _):
        step(2 * g, 0)
        step(2 * g + 1, 1)
        return 0

    lax.fori_loop(0, (nblk - 1) // 2, body, 0)
    wait_idx(nblk, 1)
    wait_data(nblk - 1, 0)
    work(nblk - 1, 0)

def _sc_gine_agg(h0c, elinc, srci, dsti, zrows):
    mesh = plsc.VectorSubcoreMesh(**_MESH)

    @functools.partial(
        pl.kernel,
        out_type=jax.ShapeDtypeStruct((4, NP, 128), jnp.float32),
        mesh=mesh,
        scratch_types=[
            [pltpu.VMEM((BA,), jnp.int32) for _ in range(2)],
            [pltpu.VMEM((BA,), jnp.int32) for _ in range(2)],
            [pltpu.VMEM((BA, 128), jnp.float32) for _ in range(2)],
            [pltpu.VMEM((BA, 128), jnp.float32) for _ in range(2)],
            pltpu.VMEM_SHARED((NP, 128), jnp.float32),
            [pltpu.SemaphoreType.DMA for _ in range(2)],
            [pltpu.SemaphoreType.DMA for _ in range(2)],
        ],
    )
    def k(h_h, e_h, s_h, d_h, z_h, o_h, idxv, didxv, gbuf, ebuf, acc, semi,
          semd):
        cid = lax.axis_index("c")
        sid = lax.axis_index("s")
        nblk = EE // 16 // BA
        for kk in range(2):
            chunk = cid * 2 + kk
            pltpu.sync_copy(z_h.at[pl.ds(sid * RPT, RPT)],
                            acc.at[pl.ds(sid * RPT, RPT)])
            plsc.subcore_barrier()

            def e_of(b):
                return sid * 10000 + jnp.minimum(b, nblk - 1) * BA

            def issue_idx(b, s):
                e0 = e_of(b)
                pltpu.async_copy(s_h.at[pl.ds(e0, BA)], idxv[s], semi[s])
                pltpu.async_copy(d_h.at[pl.ds(e0, BA)], didxv[s], semi[s])

            def wait_idx(b, s):
                e0 = e_of(b)
                pltpu.make_async_copy(s_h.at[pl.ds(e0, BA)], idxv[s],
                                      semi[s]).wait()
                pltpu.make_async_copy(d_h.at[pl.ds(e0, BA)], didxv[s],
                                      semi[s]).wait()

            def issue_data(b, s):
                e0 = e_of(b)
                pltpu.async_copy(e_h.at[chunk].at[pl.ds(e0, BA)], ebuf[s],
                                 semd[s])
                pltpu.async_copy(h_h.at[chunk].at[idxv[s]], gbuf[s],
                                 semd[s])

            def wait_data(b, s):
                e0 = e_of(b)
                pltpu.make_async_copy(e_h.at[chunk].at[pl.ds(e0, BA)],
                                      ebuf[s], semd[s]).wait()
                pltpu.make_async_copy(h_h.at[chunk].at[idxv[s]], gbuf[s],
                                      semd[s]).wait()

            def work(b, s):
                def rbody(r, _):
                    for v in range(8):
                        sl = pl.ds(v * 16, 16)
                        gbuf[s][r, sl] = jnp.maximum(
                            gbuf[s][r, sl] + ebuf[s][r, sl], 0.0)
                    return 0

                lax.fori_loop(0, BA, rbody, 0)
                pltpu.sync_copy(gbuf[s], acc.at[didxv[s]], add=True)

            _pipe2(nblk, issue_idx, wait_idx, issue_data, wait_data, work)
            plsc.subcore_barrier()
            pltpu.sync_copy(acc.at[pl.ds(sid * RPT, RPT)],
                            o_h.at[chunk].at[pl.ds(sid * RPT, RPT)])
            plsc.subcore_barrier()

    return k(h0c, elinc, srci, dsti, zrows)


def _sc_gat_logits(xlg, xrg, eeg, att16, srci, dsti):
    mesh = plsc.VectorSubcoreMesh(**_MESH)
    nb = EE // 32 // BB

    @functools.partial(
        pl.kernel,
        out_type=jax.ShapeDtypeStruct((EE, 16), jnp.float32),
        mesh=mesh,
        scratch_types=[
            [pltpu.VMEM((BB,), jnp.int32) for _ in range(2)],
            [pltpu.VMEM((BB,), jnp.int32) for _ in range(2)],
            [pltpu.VMEM((BB, 128), jnp.float32) for _ in range(2)],
            [pltpu.VMEM((BB, 128), jnp.float32) for _ in range(2)],
            [pltpu.VMEM((BB, 128), jnp.float32) for _ in range(2)],
            pltpu.VMEM((16, 128), jnp.float32),
            pltpu.VMEM((BB, 16), jnp.float32),
            pltpu.VMEM((BB, 16), jnp.float32),
            [pltpu.SemaphoreType.DMA for _ in range(2)],
            [pltpu.SemaphoreType.DMA for _ in range(2)],
        ],
    )
    def k(xl_h, xr_h, ee_h, att_h, s_h, d_h, o_h, idxv, didxv, xlb, xrb,
          eeb, attb, lbuf, pacc, semi, semd):
        cid = lax.axis_index("c")
        sid = lax.axis_index("s")
        wid = cid * 16 + sid
        pltpu.sync_copy(att_h, attb)
        lanes = lax.iota(jnp.int32, 16)

        def e_of(b):
            return wid * (EE // 32) + jnp.minimum(b, nb - 1) * BB

        def issue_idx(b, si):
            e0 = e_of(b)
            pltpu.async_copy(s_h.at[pl.ds(e0, BB)], idxv[si], semi[si])
            pltpu.async_copy(d_h.at[pl.ds(e0, BB)], didxv[si], semi[si])

        def wait_idx(b, si):
            e0 = e_of(b)
            pltpu.make_async_copy(s_h.at[pl.ds(e0, BB)], idxv[si],
                                  semi[si]).wait()
            pltpu.make_async_copy(d_h.at[pl.ds(e0, BB)], didxv[si],
                                  semi[si]).wait()

        def issue_data(b, g, si, s):
            e0 = e_of(b)
            pltpu.async_copy(xl_h.at[g].at[idxv[si]], xlb[s], semd[s])
            pltpu.async_copy(xr_h.at[g].at[didxv[si]], xrb[s], semd[s])
            pltpu.async_copy(ee_h.at[g].at[pl.ds(e0, BB)], eeb[s], semd[s])

        def wait_data(b, g, si, s):
            e0 = e_of(b)
            pltpu.make_async_copy(xl_h.at[g].at[idxv[si]], xlb[s],
                                  semd[s]).wait()
            pltpu.make_async_copy(xr_h.at[g].at[didxv[si]], xrb[s],
                                  semd[s]).wait()
            pltpu.make_async_copy(ee_h.at[g].at[pl.ds(e0, BB)], eeb[s],
                                  semd[s]).wait()

        def compute(h, c, s):
            def rbody(r, _):
                accv = jnp.zeros((16,), jnp.float32)
                for v in range(8):
                    sl = pl.ds(v * 16, 16)
                    t = xlb[s][r, sl] + xrb[s][r, sl] + eeb[s][r, sl]
                    m = jnp.maximum(t, 0.0) + 0.2 * jnp.minimum(t, 0.0)
                    accv = accv + m * attb[h * 4 + c, sl]
                if c == 0:
                    pacc[r, pl.ds(0, 16)] = accv
                elif c < 3:
                    pacc[r, pl.ds(0, 16)] = pacc[r, pl.ds(0, 16)] + accv
                else:
                    accv = accv + pacc[r, pl.ds(0, 16)]
                    for sh in (8, 4, 2, 1):
                        accv = accv + accv.at[lanes ^ sh].get(
                            mode="promise_in_bounds")
                    if h == 0:
                        row = jnp.where(lanes == 0, accv, 0.0)
                    else:
                        row = jnp.where(lanes == h, accv,
                                        lbuf[r, pl.ds(0, 16)])
                    lbuf[r, pl.ds(0, 16)] = row
                return 0

            lax.fori_loop(0, BB, rbody, 0)

        def block_steps(b, si, last):
            for h in range(HH):
                for c in range(4):
                    g = h * 4 + c
                    s = c % 2
                    wait_data(b, g, si, s)
                    if g == 0 and not last:
                        issue_idx(b + 1, 1 - si)
                    if g < 15:
                        issue_data(b, g + 1, si, 1 - s)
                    elif not last:
                        wait_idx(b + 1, 1 - si)
                        issue_data(b + 1, 0, 1 - si, 0)
                    compute(h, c, s)
            pltpu.sync_copy(lbuf, o_h.at[pl.ds(e_of(b), BB)])

        issue_idx(0, 0)
        wait_idx(0, 0)
        issue_data(0, 0, 0, 0)

        def body(g, _):
            block_steps(2 * g, 0, False)
            block_steps(2 * g + 1, 1, False)
            return 0

        lax.fori_loop(0, (nb - 1) // 2, body, 0)
        block_steps(nb - 1, 0, True)

    return k(xlg, xrg, eeg, att16, srci, dsti)


def _sc_denom(ex16, dsti, zrows16):
    mesh = plsc.VectorSubcoreMesh(**_MESH)
    nb = EE // 32 // BB

    @functools.partial(
        pl.kernel,
        out_type=jax.ShapeDtypeStruct((2, NP, 16), jnp.float32),
        mesh=mesh,
        scratch_types=[
            pltpu.VMEM((BB,), jnp.int32),
            pltpu.VMEM((BB, 16), jnp.float32),
            pltpu.VMEM_SHARED((NP, 16), jnp.float32),
            pltpu.SemaphoreType.DMA,
        ],
    )
    def k(ex_h, d_h, z_h, o_h, didxv, exb, acc, sem):
        cid = lax.axis_index("c")
        sid = lax.axis_index("s")
        pltpu.sync_copy(z_h.at[pl.ds(sid * RPT, RPT)],
                        acc.at[pl.ds(sid * RPT, RPT)])
        plsc.subcore_barrier()

        def blk(b, _):
            e0 = (cid * 16 + sid) * (EE // 32) + b * BB
            c1 = pltpu.async_copy(d_h.at[pl.ds(e0, BB)], didxv, sem)
            c2 = pltpu.async_copy(ex_h.at[pl.ds(e0, BB)], exb, sem)
            c1.wait()
            c2.wait()
            pltpu.sync_copy(exb, acc.at[didxv], add=True)
            return 0

        lax.fori_loop(0, nb, blk, 0)
        plsc.subcore_barrier()
        pltpu.sync_copy(acc.at[pl.ds(sid * RPT, RPT)],
                        o_h.at[cid].at[pl.ds(sid * RPT, RPT)])

    return k(ex16, dsti, zrows16)


def _sc_gat_out(xlg, exh, srci, dsti, zrows):
    mesh = plsc.VectorSubcoreMesh(**_MESH)

    @functools.partial(
        pl.kernel,
        out_type=jax.ShapeDtypeStruct((16, NP, 128), jnp.float32),
        mesh=mesh,
        scratch_types=[
            [pltpu.VMEM((BA,), jnp.int32) for _ in range(2)],
            [pltpu.VMEM((BA,), jnp.int32) for _ in range(2)],
            [pltpu.VMEM((BA, 128), jnp.float32) for _ in range(2)],
            [pltpu.VMEM((BA, 16), jnp.float32) for _ in range(2)],
            pltpu.VMEM_SHARED((NP, 128), jnp.float32),
            [pltpu.SemaphoreType.DMA for _ in range(2)],
            [pltpu.SemaphoreType.DMA for _ in range(2)],
        ],
    )
    def k(xl_h, ex_h, s_h, d_h, z_h, o_h, idxv, didxv, gbuf, exb, acc, semi,
          semd):
        cid = lax.axis_index("c")
        sid = lax.axis_index("s")
        nblk = EE // 16 // BA
        for g8 in range(8):
            g = cid * 8 + g8
            h = cid * 2 + (g8 // 4)
            pltpu.sync_copy(z_h.at[pl.ds(sid * RPT, RPT)],
                            acc.at[pl.ds(sid * RPT, RPT)])
            plsc.subcore_barrier()

            def e_of(b):
                return sid * 10000 + jnp.minimum(b, nblk - 1) * BA

            def issue_idx(b, s):
                e0 = e_of(b)
                pltpu.async_copy(s_h.at[pl.ds(e0, BA)], idxv[s], semi[s])
                pltpu.async_copy(d_h.at[pl.ds(e0, BA)], didxv[s], semi[s])

            def wait_idx(b, s):
                e0 = e_of(b)
                pltpu.make_async_copy(s_h.at[pl.ds(e0, BA)], idxv[s],
                                      semi[s]).wait()
                pltpu.make_async_copy(d_h.at[pl.ds(e0, BA)], didxv[s],
                                      semi[s]).wait()

            def issue_data(b, s):
                e0 = e_of(b)
                pltpu.async_copy(ex_h.at[h].at[pl.ds(e0, BA)], exb[s],
                                 semd[s])
                pltpu.async_copy(xl_h.at[g].at[idxv[s]], gbuf[s], semd[s])

            def wait_data(b, s):
                e0 = e_of(b)
                pltpu.make_async_copy(ex_h.at[h].at[pl.ds(e0, BA)], exb[s],
                                      semd[s]).wait()
                pltpu.make_async_copy(xl_h.at[g].at[idxv[s]], gbuf[s],
                                      semd[s]).wait()

            def work(b, s):
                def rbody(r, _):
                    av = exb[s][r, pl.ds(0, 16)]
                    for v in range(8):
                        sl = pl.ds(v * 16, 16)
                        gbuf[s][r, sl] = gbuf[s][r, sl] * av
                    return 0

                lax.fori_loop(0, BA, rbody, 0)
                pltpu.sync_copy(gbuf[s], acc.at[didxv[s]], add=True)

            _pipe2(nblk, issue_idx, wait_idx, issue_data, wait_data, work)
            plsc.subcore_barrier()
            pltpu.sync_copy(acc.at[pl.ds(sid * RPT, RPT)],
                            o_h.at[g].at[pl.ds(sid * RPT, RPT)])
            plsc.subcore_barrier()

    return k(xlg, exh, srci, dsti, zrows)


def _sc_gcn_agg(xwc, srci, dsti, zrows):
    mesh = plsc.VectorSubcoreMesh(**_MESH)

    @functools.partial(
        pl.kernel,
        out_type=jax.ShapeDtypeStruct((4, NP, 128), jnp.float32),
        mesh=mesh,
        scratch_types=[
            [pltpu.VMEM((BA,), jnp.int32) for _ in range(2)],
            [pltpu.VMEM((BA,), jnp.int32) for _ in range(2)],
            [pltpu.VMEM((BA, 128), jnp.float32) for _ in range(2)],
            pltpu.VMEM_SHARED((NP, 128), jnp.float32),
            [pltpu.SemaphoreType.DMA for _ in range(2)],
            [pltpu.SemaphoreType.DMA for _ in range(2)],
        ],
    )
    def k(xw_h, s_h, d_h, z_h, o_h, idxv, didxv, gbuf, acc, semi, semd):
        cid = lax.axis_index("c")
        sid = lax.axis_index("s")
        nblk = EE // 16 // BA
        for kk in range(2):
            chunk = cid * 2 + kk
            pltpu.sync_copy(z_h.at[pl.ds(sid * RPT, RPT)],
                            acc.at[pl.ds(sid * RPT, RPT)])
            plsc.subcore_barrier()

            def e_of(b):
                return sid * 10000 + jnp.minimum(b, nblk - 1) * BA

            def issue_idx(b, s):
                e0 = e_of(b)
                pltpu.async_copy(s_h.at[pl.ds(e0, BA)], idxv[s], semi[s])
                pltpu.async_copy(d_h.at[pl.ds(e0, BA)], didxv[s], semi[s])

            def wait_idx(b, s):
                e0 = e_of(b)
                pltpu.make_async_copy(s_h.at[pl.ds(e0, BA)], idxv[s],
                                      semi[s]).wait()
                pltpu.make_async_copy(d_h.at[pl.ds(e0, BA)], didxv[s],
                                      semi[s]).wait()

            def issue_data(b, s):
                pltpu.async_copy(xw_h.at[chunk].at[idxv[s]], gbuf[s],
                                 semd[s])

            def wait_data(b, s):
                pltpu.make_async_copy(xw_h.at[chunk].at[idxv[s]], gbuf[s],
                                      semd[s]).wait()

            def work(b, s):
                pltpu.sync_copy(gbuf[s], acc.at[didxv[s]], add=True)

            _pipe2(nblk, issue_idx, wait_idx, issue_data, wait_data, work)
            plsc.subcore_barrier()
            pltpu.sync_copy(acc.at[pl.ds(sid * RPT, RPT)],
                            o_h.at[chunk].at[pl.ds(sid * RPT, RPT)])
            plsc.subcore_barrier()

    return k(xwc, srci, dsti, zrows)


def kernel(x, edge_index, edge_attr, params):
    p = params
    n, in_f = x.shape
    src = edge_index[0]
    dst = edge_index[1]

    xp = jnp.zeros((NP, 128), jnp.float32).at[:n, :in_f].set(x)
    inwT = jnp.zeros((128, DD), jnp.float32).at[:in_f, :].set(p['in_w'].T)
    inb4 = p['in_b'].reshape(4, 1, 128)
    ea8 = jnp.zeros((EE, 8), jnp.float32).at[:, :4].set(edge_attr)
    gewT = jnp.zeros((8, DD), jnp.float32).at[:4, :].set(p['gine_edge_w'].T)
    geb4 = p['gine_edge_b'].reshape(4, 1, 128)
    gwT = jnp.zeros((8, HH * DD), jnp.float32).at[:4, :].set(p['gat_we'].T)
    zb16 = jnp.zeros((16, 1, 128), jnp.float32)
    att16 = p['gat_att'].reshape(16, 128)
    zrows = jnp.zeros((NP, 128), jnp.float32)
    zrows16 = jnp.zeros((NP, 16), jnp.float32)

    h0c = _tc_inproj(xp, inwT, inb4)

    elinc = _tc_edge_mm(ea8, gewT, geb4, 4)
    agg4 = _sc_gine_agg(h0c, elinc, src, dst, zrows)
    h1c = _tc_gine_mlp(h0c, agg4, p['gine_w1'].T, p['gine_b1'][None],
                       p['gine_w2'].T, p['gine_b2'][None],
                       p['ln0_g'][None], p['ln0_b'][None])

    xlg, xrg = _tc_gat_lin(h1c, p['gat_wl'].T, p['gat_wr'].T,
                           p['gat_bl'].reshape(16, 1, 128),
                           p['gat_br'].reshape(16, 1, 128))
    eeg = _tc_edge_mm(ea8, gwT, zb16, 16)
    logit16 = _sc_gat_logits(xlg, xrg, eeg, att16, src, dst)
    cmax = _tc_logit_max(logit16)
    ex16, exh = _tc_exp(logit16, cmax)
    denom2 = _sc_denom(ex16, dst, zrows16)
    rcph, dinv16 = _tc_denom_finish(denom2)
    gat16 = _sc_gat_out(xlg, exh, src, dst, zrows)
    h2c, xw2c = _tc_gat_finish(gat16, rcph, dinv16, h1c, p['gat_bias'][None],
                               p['ln1_g'][None], p['ln1_b'][None],
                               p['gcn2_w'].T)

    agg2 = _sc_gcn_agg(xw2c, src, dst, zrows)
    h3c, xw3c = _tc_gcn_finish(agg2, xw2c, dinv16, h2c, p['gcn2_b'][None],
                               p['ln2_g'][None], p['ln2_b'][None],
                               p['gcn3_w'].T, True)
    agg3 = _sc_gcn_agg(xw3c, src, dst, zrows)
    h4c, _ = _tc_gcn_finish(agg3, xw3c, dinv16, h3c, p['gcn3_b'][None],
                            p['ln3_g'][None], p['ln3_b'][None],
                            p['gcn3_w'].T, False)

    return jnp.concatenate([h4c[c][:n] for c in range(4)], axis=-1)
